# baseline (device time: 75517 ns/iter reference)
import jax
import jax.numpy as jnp
from jax import lax
from jax.experimental import pallas as pl
from jax.experimental.pallas import tpu as pltpu

N_DEV = 16
DH = 128
SCALE = 0.08838834764831843


def _deint(W, Hl, scale=1.0):
    D = W.shape[0]
    return ((W * scale).astype(jnp.bfloat16)
            .reshape(D, Hl, DH // 2, 2)
            .transpose(0, 1, 3, 2)
            .reshape(D, Hl * DH))


def _fused_attn_allreduce(x, wqp, wkp, wv, wo, cos, sin, B, Hl):
    M, D = x.shape
    N = wo.shape[1]
    Sq = M // B
    half = DH // 2
    Q4 = M // 4
    S16 = M // 16

    def body(x_ref, wqp_ref, wkp_ref, wv_ref, wo_ref, cos_ref, sin_ref,
             out_ref, k_scr, v_scr, land_p, land_z,
             rsp_send, rsp_recv, rsz_send, rsz_recv,
             agz_send, agz_recv, agp_send, agp_recv):
        MESH = pl.DeviceIdType.MESH
        me = lax.axis_index("i")
        q = me & 3
        r = me >> 2

        plane_peers = [me ^ t for t in (1, 2, 3)]
        z_peers = [me ^ (t << 2) for t in (1, 2, 3)]

        barrier = pltpu.get_barrier_semaphore()
        for peer in plane_peers + z_peers:
            pl.semaphore_signal(barrier, inc=1, device_id=(peer,),
                                device_id_type=pl.DeviceIdType.MESH)

        def rope(t, c, s):
            t1, t2 = t[:, :half], t[:, half:]
            return jnp.concatenate([t1 * c - t2 * s, t2 * c + t1 * s],
                                   axis=1)

        cos_b = cos_ref[...]
        sin_b = sin_ref[...]
        for b in range(B):
            rows = slice(b * Sq, (b + 1) * Sq)
            xb = x_ref[rows, :]
            kfull = jnp.dot(xb, wkp_ref[...],
                            preferred_element_type=jnp.float32
                            ).astype(jnp.bfloat16)
            for h in range(Hl):
                cols = slice(h * DH, (h + 1) * DH)
                k_scr[rows, cols] = rope(kfull[:, cols], cos_b, sin_b)
            v_scr[rows, :] = jnp.dot(xb, wv_ref[...],
                                     preferred_element_type=jnp.float32
                                     ).astype(jnp.bfloat16)

        pl.semaphore_wait(barrier, 6)

        def compute_quarter(idx):
            rows = pl.ds(idx * Q4, Q4)
            b0 = (idx >> 1) * Sq
            s0 = (idx & 1) * Q4
            xq = x_ref[rows, :]
            cos_q = cos_ref[pl.ds(s0, Q4), :]
            sin_q = sin_ref[pl.ds(s0, Q4), :]
            qfull = jnp.dot(xq, wqp_ref[...],
                            preferred_element_type=jnp.float32
                            ).astype(jnp.bfloat16)
            parts = []
            for h in range(Hl):
                cols = slice(h * DH, (h + 1) * DH)
                qh = rope(qfull[:, cols], cos_q, sin_q)
                kh = k_scr[pl.ds(b0, Sq), cols]
                vh = v_scr[pl.ds(b0, Sq), cols]
                s = lax.dot_general(
                    qh, kh, (((1,), (1,)), ((), ())),
                    preferred_element_type=jnp.float32)
                e = jnp.exp(s)
                recip = 1.0 / jnp.sum(e, axis=1, keepdims=True)
                ctx_h = jnp.dot(e.astype(jnp.bfloat16), vh,
                                preferred_element_type=jnp.float32)
                parts.append((ctx_h * recip).astype(jnp.bfloat16))
            ctx_q = jnp.concatenate(parts, axis=1)
            out_ref[rows, :] = jnp.dot(
                ctx_q, wo_ref[...], preferred_element_type=jnp.float32
            ).astype(jnp.bfloat16)

        rd = {}
        for t in (1, 2, 3):
            compute_quarter(q ^ t)
            for j in range(4):
                zsub = r ^ ((j + 1) % 4)
                rdma = pltpu.make_async_remote_copy(
                    src_ref=out_ref.at[
                        pl.ds((q ^ t) * Q4 + zsub * S16, S16), :],
                    dst_ref=land_p.at[t - 1, j],
                    send_sem=rsp_send.at[t - 1, j],
                    recv_sem=rsp_recv.at[t - 1, j],
                    device_id=(plane_peers[t - 1],),
                    device_id_type=pl.DeviceIdType.MESH,
                )
                rdma.start()
                rd[(t, j)] = rdma
        compute_quarter(q)
        myq = pl.ds(q * Q4, Q4)
        mine = pl.ds(q * Q4 + r * S16, S16)

        z_rdmas = []
        for j in range(4):
            zsub = r ^ ((j + 1) % 4)
            rows_j = pl.ds(q * Q4 + zsub * S16, S16)
            for t in (1, 2, 3):
                rd[(t, j)].wait()
            out_ref[rows_j, :] = (out_ref[rows_j, :] + land_p[0, j]
                                  + land_p[1, j] + land_p[2, j])
            if j < 3:
                tz = j + 1
                rdma = pltpu.make_async_remote_copy(
                    src_ref=out_ref.at[rows_j, :],
                    dst_ref=land_z.at[tz - 1],
                    send_sem=rsz_send.at[tz - 1],
                    recv_sem=rsz_recv.at[tz - 1],
                    device_id=(z_peers[tz - 1],),
                    device_id_type=pl.DeviceIdType.MESH,
                )
                rdma.start()
                z_rdmas.append(rdma)
        for rdma in z_rdmas:
            rdma.wait()
        out_ref[mine, :] = (out_ref[mine, :] + land_z[0]
                            + land_z[1] + land_z[2])

        agz = []
        for t in (1, 2, 3):
            rdma = pltpu.make_async_remote_copy(
                src_ref=out_ref.at[mine, :],
                dst_ref=out_ref.at[mine, :],
                send_sem=agz_send.at[t - 1],
                recv_sem=agz_recv.at[t - 1],
                device_id=(z_peers[t - 1],),
                device_id_type=MESH,
            )
            rdma.start()
            agz.append(rdma)

        def forward_block(u, zsub):
            rows = pl.ds(q * Q4 + zsub * S16, S16)
            started = []
            for t in (1, 2, 3):
                rdma = pltpu.make_async_remote_copy(
                    src_ref=out_ref.at[rows, :],
                    dst_ref=out_ref.at[rows, :],
                    send_sem=agp_send.at[t - 1, u],
                    recv_sem=agp_recv.at[t - 1, u],
                    device_id=(plane_peers[t - 1],),
                    device_id_type=MESH,
                )
                rdma.start()
                started.append(rdma)
            return started

        agp = forward_block(0, r)
        for u in (1, 2, 3):
            agz[u - 1].wait_recv()
            agp += forward_block(u, r ^ u)
        for rdma in agz:
            rdma.wait_send()
        for rdma in agp:
            rdma.wait()

        def _exit(second_barrier):
            for peer in plane_peers + z_peers:
                pl.semaphore_signal(second_barrier, inc=1,
                                    device_id=(peer,),
                                    device_id_type=pl.DeviceIdType.MESH)
            pl.semaphore_wait(second_barrier, 6)

        pl.run_scoped(_exit, second_barrier=pltpu.SemaphoreType.REGULAR)

    return pl.pallas_call(
        body,
        out_shape=jax.ShapeDtypeStruct((M, N), jnp.bfloat16),
        in_specs=[pl.BlockSpec(memory_space=pltpu.VMEM)] * 7,
        out_specs=pl.BlockSpec(memory_space=pltpu.VMEM),
        scratch_shapes=[
            pltpu.VMEM((M, Hl * DH), jnp.bfloat16),
            pltpu.VMEM((M, Hl * DH), jnp.bfloat16),
            pltpu.VMEM((3, 4, S16, N), jnp.bfloat16),
            pltpu.VMEM((3, S16, N), jnp.bfloat16),
            pltpu.SemaphoreType.DMA((3, 4)),
            pltpu.SemaphoreType.DMA((3, 4)),
            pltpu.SemaphoreType.DMA((3,)),
            pltpu.SemaphoreType.DMA((3,)),
            pltpu.SemaphoreType.DMA((3,)),
            pltpu.SemaphoreType.DMA((3,)),
            pltpu.SemaphoreType.DMA((3, 4)),
            pltpu.SemaphoreType.DMA((3, 4)),
        ],
        compiler_params=pltpu.CompilerParams(collective_id=0),
    )(x, wqp, wkp, wv, wo, cos, sin)


def kernel(x, Wq, Wk, Wv, Wo):
    B, Sq, D = x.shape
    Hl = Wq.shape[1] // DH
    xb = x.astype(jnp.bfloat16).reshape(B * Sq, D)

    inv = 1.0 / (10000.0 ** (jnp.arange(0, DH, 2, dtype=jnp.float32) / DH))
    ang = jnp.arange(Sq, dtype=jnp.float32)[:, None] * inv[None, :]
    cos = jnp.cos(ang).astype(jnp.bfloat16)
    sin = jnp.sin(ang).astype(jnp.bfloat16)

    out = _fused_attn_allreduce(
        xb, _deint(Wq, Hl, scale=SCALE), _deint(Wk, Hl),
        Wv.astype(jnp.bfloat16), Wo.astype(jnp.bfloat16),
        cos, sin, B, Hl)
    return out.astype(jnp.float32).reshape(B, Sq, D)


# device time: 71173 ns/iter; 1.0610x vs baseline; 1.0610x over previous
import jax
import jax.numpy as jnp
from jax import lax
from jax.experimental import pallas as pl
from jax.experimental.pallas import tpu as pltpu

N_DEV = 16
DH = 128
SCALE = 0.08838834764831843


def _deint(W, Hl, scale=1.0):
    D = W.shape[0]
    return ((W * scale).astype(jnp.bfloat16)
            .reshape(D, Hl, DH // 2, 2)
            .transpose(0, 1, 3, 2)
            .reshape(D, Hl * DH))


def _fused_attn_allreduce(x, wqp, wkp, wv, wo, cos, sin, B, Hl):
    M, D = x.shape
    N = wo.shape[1]
    Sq = M // B
    half = DH // 2
    Q4 = M // 4
    S16 = M // 16

    def body(x_ref, wqp_ref, wkp_ref, wv_ref, wo_ref, cos_ref, sin_ref,
             out_ref, k_scr, v_scr, land_p, land_z,
             rsp_send, rsp_recv, rsz_send, rsz_recv,
             agz_send, agz_recv, agp_send, agp_recv):
        MESH = pl.DeviceIdType.MESH
        me = lax.axis_index("i")
        q = me & 3
        r = me >> 2

        plane_peers = [me ^ t for t in (1, 2, 3)]
        z_peers = [me ^ (t << 2) for t in (1, 2, 3)]

        barrier = pltpu.get_barrier_semaphore()
        for peer in plane_peers + z_peers:
            pl.semaphore_signal(barrier, inc=1, device_id=(peer,),
                                device_id_type=pl.DeviceIdType.MESH)

        def rope(t, c, s):
            t1, t2 = t[:, :half], t[:, half:]
            return jnp.concatenate([t1 * c - t2 * s, t2 * c + t1 * s],
                                   axis=1)

        cos_b = cos_ref[...]
        sin_b = sin_ref[...]
        for b in range(B):
            rows = slice(b * Sq, (b + 1) * Sq)
            xb = x_ref[rows, :]
            kfull = jnp.dot(xb, wkp_ref[...],
                            preferred_element_type=jnp.float32
                            ).astype(jnp.bfloat16)
            for h in range(Hl):
                cols = slice(h * DH, (h + 1) * DH)
                k_scr[rows, cols] = rope(kfull[:, cols], cos_b, sin_b)
            v_scr[rows, :] = jnp.dot(xb, wv_ref[...],
                                     preferred_element_type=jnp.float32
                                     ).astype(jnp.bfloat16)

        pl.semaphore_wait(barrier, 6)

        def compute_batch(bidx):
            rows = pl.ds(bidx * Sq, Sq)
            xq = x_ref[rows, :]
            qfull = jnp.dot(xq, wqp_ref[...],
                            preferred_element_type=jnp.float32
                            ).astype(jnp.bfloat16)
            parts = []
            for h in range(Hl):
                cols = slice(h * DH, (h + 1) * DH)
                qh = rope(qfull[:, cols], cos_b, sin_b)
                kh = k_scr[rows, cols]
                vh = v_scr[rows, cols]
                s = lax.dot_general(
                    qh, kh, (((1,), (1,)), ((), ())),
                    preferred_element_type=jnp.float32)
                e = jnp.exp(s)
                recip = 1.0 / jnp.sum(e, axis=1, keepdims=True)
                ctx_h = jnp.dot(e.astype(jnp.bfloat16), vh,
                                preferred_element_type=jnp.float32)
                parts.append((ctx_h * recip).astype(jnp.bfloat16))
            ctx_b = jnp.concatenate(parts, axis=1)
            out_ref[rows, :] = jnp.dot(
                ctx_b, wo_ref[...], preferred_element_type=jnp.float32
            ).astype(jnp.bfloat16)

        def start_subsends(t):
            for j in range(4):
                zsub = r ^ ((j + 1) % 4)
                rdma = pltpu.make_async_remote_copy(
                    src_ref=out_ref.at[
                        pl.ds((q ^ t) * Q4 + zsub * S16, S16), :],
                    dst_ref=land_p.at[t - 1, j],
                    send_sem=rsp_send.at[t - 1, j],
                    recv_sem=rsp_recv.at[t - 1, j],
                    device_id=(plane_peers[t - 1],),
                    device_id_type=pl.DeviceIdType.MESH,
                )
                rdma.start()
                rd[(t, j)] = rdma

        rd = {}
        compute_batch((q >> 1) ^ 1)
        start_subsends(2)
        start_subsends(3)
        compute_batch(q >> 1)
        start_subsends(1)
        myq = pl.ds(q * Q4, Q4)
        mine = pl.ds(q * Q4 + r * S16, S16)

        z_rdmas = []
        for j in range(4):
            zsub = r ^ ((j + 1) % 4)
            rows_j = pl.ds(q * Q4 + zsub * S16, S16)
            for t in (1, 2, 3):
                rd[(t, j)].wait()
            out_ref[rows_j, :] = (out_ref[rows_j, :] + land_p[0, j]
                                  + land_p[1, j] + land_p[2, j])
            if j < 3:
                tz = j + 1
                rdma = pltpu.make_async_remote_copy(
                    src_ref=out_ref.at[rows_j, :],
                    dst_ref=land_z.at[tz - 1],
                    send_sem=rsz_send.at[tz - 1],
                    recv_sem=rsz_recv.at[tz - 1],
                    device_id=(z_peers[tz - 1],),
                    device_id_type=pl.DeviceIdType.MESH,
                )
                rdma.start()
                z_rdmas.append(rdma)
        for rdma in z_rdmas:
            rdma.wait()
        out_ref[mine, :] = (out_ref[mine, :] + land_z[0]
                            + land_z[1] + land_z[2])

        agz = []
        for t in (1, 2, 3):
            rdma = pltpu.make_async_remote_copy(
                src_ref=out_ref.at[mine, :],
                dst_ref=out_ref.at[mine, :],
                send_sem=agz_send.at[t - 1],
                recv_sem=agz_recv.at[t - 1],
                device_id=(z_peers[t - 1],),
                device_id_type=MESH,
            )
            rdma.start()
            agz.append(rdma)

        def forward_block(u, zsub):
            rows = pl.ds(q * Q4 + zsub * S16, S16)
            started = []
            for t in (1, 2, 3):
                rdma = pltpu.make_async_remote_copy(
                    src_ref=out_ref.at[rows, :],
                    dst_ref=out_ref.at[rows, :],
                    send_sem=agp_send.at[t - 1, u],
                    recv_sem=agp_recv.at[t - 1, u],
                    device_id=(plane_peers[t - 1],),
                    device_id_type=MESH,
                )
                rdma.start()
                started.append(rdma)
            return started

        agp = forward_block(0, r)
        for u in (1, 2, 3):
            agz[u - 1].wait_recv()
            agp += forward_block(u, r ^ u)
        for rdma in agz:
            rdma.wait_send()
        for rdma in agp:
            rdma.wait()

        def _exit(second_barrier):
            for peer in plane_peers + z_peers:
                pl.semaphore_signal(second_barrier, inc=1,
                                    device_id=(peer,),
                                    device_id_type=pl.DeviceIdType.MESH)
            pl.semaphore_wait(second_barrier, 6)

        pl.run_scoped(_exit, second_barrier=pltpu.SemaphoreType.REGULAR)

    return pl.pallas_call(
        body,
        out_shape=jax.ShapeDtypeStruct((M, N), jnp.bfloat16),
        in_specs=[pl.BlockSpec(memory_space=pltpu.VMEM)] * 7,
        out_specs=pl.BlockSpec(memory_space=pltpu.VMEM),
        scratch_shapes=[
            pltpu.VMEM((M, Hl * DH), jnp.bfloat16),
            pltpu.VMEM((M, Hl * DH), jnp.bfloat16),
            pltpu.VMEM((3, 4, S16, N), jnp.bfloat16),
            pltpu.VMEM((3, S16, N), jnp.bfloat16),
            pltpu.SemaphoreType.DMA((3, 4)),
            pltpu.SemaphoreType.DMA((3, 4)),
            pltpu.SemaphoreType.DMA((3,)),
            pltpu.SemaphoreType.DMA((3,)),
            pltpu.SemaphoreType.DMA((3,)),
            pltpu.SemaphoreType.DMA((3,)),
            pltpu.SemaphoreType.DMA((3, 4)),
            pltpu.SemaphoreType.DMA((3, 4)),
        ],
        compiler_params=pltpu.CompilerParams(collective_id=0),
    )(x, wqp, wkp, wv, wo, cos, sin)


def kernel(x, Wq, Wk, Wv, Wo):
    B, Sq, D = x.shape
    Hl = Wq.shape[1] // DH
    xb = x.astype(jnp.bfloat16).reshape(B * Sq, D)

    inv = 1.0 / (10000.0 ** (jnp.arange(0, DH, 2, dtype=jnp.float32) / DH))
    ang = jnp.arange(Sq, dtype=jnp.float32)[:, None] * inv[None, :]
    cos = jnp.cos(ang).astype(jnp.bfloat16)
    sin = jnp.sin(ang).astype(jnp.bfloat16)

    out = _fused_attn_allreduce(
        xb, _deint(Wq, Hl, scale=SCALE), _deint(Wk, Hl),
        Wv.astype(jnp.bfloat16), Wo.astype(jnp.bfloat16),
        cos, sin, B, Hl)
    return out.astype(jnp.float32).reshape(B, Sq, D)


# device time: 71145 ns/iter; 1.0615x vs baseline; 1.0004x over previous
import jax
import jax.numpy as jnp
from jax import lax
from jax.experimental import pallas as pl
from jax.experimental.pallas import tpu as pltpu

N_DEV = 16
DH = 128
SCALE = 0.08838834764831843


def _deint(W, Hl, scale=1.0):
    D = W.shape[0]
    return ((W * scale).astype(jnp.bfloat16)
            .reshape(D, Hl, DH // 2, 2)
            .transpose(0, 1, 3, 2)
            .reshape(D, Hl * DH))


def _fused_attn_allreduce(x, wqp, wkp, wv, wo, cos, sin, B, Hl):
    M, D = x.shape
    N = wo.shape[1]
    Sq = M // B
    half = DH // 2
    Q4 = M // 4
    S16 = M // 16

    def body(x_ref, wqp_ref, wkp_ref, wv_ref, wo_ref, cos_ref, sin_ref,
             out_ref, k_scr, v_scr, land_p, land_z,
             rsp_send, rsp_recv, rsz_send, rsz_recv,
             agz_send, agz_recv, agp_send, agp_recv):
        MESH = pl.DeviceIdType.MESH
        me = lax.axis_index("i")
        q = me & 3
        r = me >> 2

        plane_peers = [me ^ t for t in (1, 2, 3)]
        z_peers = [me ^ (t << 2) for t in (1, 2, 3)]

        barrier = pltpu.get_barrier_semaphore()
        for peer in plane_peers + z_peers:
            pl.semaphore_signal(barrier, inc=1, device_id=(peer,),
                                device_id_type=pl.DeviceIdType.MESH)

        def rope(t, c, s):
            t1, t2 = t[:, :half], t[:, half:]
            return jnp.concatenate([t1 * c - t2 * s, t2 * c + t1 * s],
                                   axis=1)

        cos_b = cos_ref[...]
        sin_b = sin_ref[...]
        kfull = jnp.dot(x_ref[...], wkp_ref[...],
                        preferred_element_type=jnp.float32
                        ).astype(jnp.bfloat16)
        for b in range(B):
            rows = slice(b * Sq, (b + 1) * Sq)
            for h in range(Hl):
                cols = slice(h * DH, (h + 1) * DH)
                k_scr[rows, cols] = rope(kfull[rows, cols], cos_b, sin_b)
        v_scr[...] = jnp.dot(x_ref[...], wv_ref[...],
                             preferred_element_type=jnp.float32
                             ).astype(jnp.bfloat16)

        pl.semaphore_wait(barrier, 6)

        def compute_batch(bidx):
            rows = pl.ds(bidx * Sq, Sq)
            xq = x_ref[rows, :]
            qfull = jnp.dot(xq, wqp_ref[...],
                            preferred_element_type=jnp.float32
                            ).astype(jnp.bfloat16)
            parts = []
            for h in range(Hl):
                cols = slice(h * DH, (h + 1) * DH)
                qh = rope(qfull[:, cols], cos_b, sin_b)
                kh = k_scr[rows, cols]
                vh = v_scr[rows, cols]
                s = lax.dot_general(
                    qh, kh, (((1,), (1,)), ((), ())),
                    preferred_element_type=jnp.float32)
                e = jnp.exp(s)
                recip = 1.0 / jnp.sum(e, axis=1, keepdims=True)
                ctx_h = jnp.dot(e.astype(jnp.bfloat16), vh,
                                preferred_element_type=jnp.float32)
                parts.append((ctx_h * recip).astype(jnp.bfloat16))
            ctx_b = jnp.concatenate(parts, axis=1)
            out_ref[rows, :] = jnp.dot(
                ctx_b, wo_ref[...], preferred_element_type=jnp.float32
            ).astype(jnp.bfloat16)

        def start_subsends(t):
            for j in range(4):
                zsub = r ^ ((j + 1) % 4)
                rdma = pltpu.make_async_remote_copy(
                    src_ref=out_ref.at[
                        pl.ds((q ^ t) * Q4 + zsub * S16, S16), :],
                    dst_ref=land_p.at[t - 1, j],
                    send_sem=rsp_send.at[t - 1, j],
                    recv_sem=rsp_recv.at[t - 1, j],
                    device_id=(plane_peers[t - 1],),
                    device_id_type=pl.DeviceIdType.MESH,
                )
                rdma.start()
                rd[(t, j)] = rdma

        rd = {}
        compute_batch((q >> 1) ^ 1)
        start_subsends(2)
        start_subsends(3)
        compute_batch(q >> 1)
        start_subsends(1)
        myq = pl.ds(q * Q4, Q4)
        mine = pl.ds(q * Q4 + r * S16, S16)

        z_rdmas = []
        for j in range(4):
            zsub = r ^ ((j + 1) % 4)
            rows_j = pl.ds(q * Q4 + zsub * S16, S16)
            for t in (1, 2, 3):
                rd[(t, j)].wait()
            out_ref[rows_j, :] = (out_ref[rows_j, :] + land_p[0, j]
                                  + land_p[1, j] + land_p[2, j])
            if j < 3:
                tz = j + 1
                rdma = pltpu.make_async_remote_copy(
                    src_ref=out_ref.at[rows_j, :],
                    dst_ref=land_z.at[tz - 1],
                    send_sem=rsz_send.at[tz - 1],
                    recv_sem=rsz_recv.at[tz - 1],
                    device_id=(z_peers[tz - 1],),
                    device_id_type=pl.DeviceIdType.MESH,
                )
                rdma.start()
                z_rdmas.append(rdma)
        for rdma in z_rdmas:
            rdma.wait()
        out_ref[mine, :] = (out_ref[mine, :] + land_z[0]
                            + land_z[1] + land_z[2])

        agz = []
        for t in (1, 2, 3):
            rdma = pltpu.make_async_remote_copy(
                src_ref=out_ref.at[mine, :],
                dst_ref=out_ref.at[mine, :],
                send_sem=agz_send.at[t - 1],
                recv_sem=agz_recv.at[t - 1],
                device_id=(z_peers[t - 1],),
                device_id_type=MESH,
            )
            rdma.start()
            agz.append(rdma)

        def forward_block(u, zsub):
            rows = pl.ds(q * Q4 + zsub * S16, S16)
            started = []
            for t in (1, 2, 3):
                rdma = pltpu.make_async_remote_copy(
                    src_ref=out_ref.at[rows, :],
                    dst_ref=out_ref.at[rows, :],
                    send_sem=agp_send.at[t - 1, u],
                    recv_sem=agp_recv.at[t - 1, u],
                    device_id=(plane_peers[t - 1],),
                    device_id_type=MESH,
                )
                rdma.start()
                started.append(rdma)
            return started

        agp = forward_block(0, r)
        for u in (1, 2, 3):
            agz[u - 1].wait_recv()
            agp += forward_block(u, r ^ u)
        for rdma in agz:
            rdma.wait_send()
        for rdma in agp:
            rdma.wait()

        def _exit(second_barrier):
            for peer in plane_peers + z_peers:
                pl.semaphore_signal(second_barrier, inc=1,
                                    device_id=(peer,),
                                    device_id_type=pl.DeviceIdType.MESH)
            pl.semaphore_wait(second_barrier, 6)

        pl.run_scoped(_exit, second_barrier=pltpu.SemaphoreType.REGULAR)

    return pl.pallas_call(
        body,
        out_shape=jax.ShapeDtypeStruct((M, N), jnp.bfloat16),
        in_specs=[pl.BlockSpec(memory_space=pltpu.VMEM)] * 7,
        out_specs=pl.BlockSpec(memory_space=pltpu.VMEM),
        scratch_shapes=[
            pltpu.VMEM((M, Hl * DH), jnp.bfloat16),
            pltpu.VMEM((M, Hl * DH), jnp.bfloat16),
            pltpu.VMEM((3, 4, S16, N), jnp.bfloat16),
            pltpu.VMEM((3, S16, N), jnp.bfloat16),
            pltpu.SemaphoreType.DMA((3, 4)),
            pltpu.SemaphoreType.DMA((3, 4)),
            pltpu.SemaphoreType.DMA((3,)),
            pltpu.SemaphoreType.DMA((3,)),
            pltpu.SemaphoreType.DMA((3,)),
            pltpu.SemaphoreType.DMA((3,)),
            pltpu.SemaphoreType.DMA((3, 4)),
            pltpu.SemaphoreType.DMA((3, 4)),
        ],
        compiler_params=pltpu.CompilerParams(collective_id=0),
    )(x, wqp, wkp, wv, wo, cos, sin)


def kernel(x, Wq, Wk, Wv, Wo):
    B, Sq, D = x.shape
    Hl = Wq.shape[1] // DH
    xb = x.astype(jnp.bfloat16).reshape(B * Sq, D)

    inv = 1.0 / (10000.0 ** (jnp.arange(0, DH, 2, dtype=jnp.float32) / DH))
    ang = jnp.arange(Sq, dtype=jnp.float32)[:, None] * inv[None, :]
    cos = jnp.cos(ang).astype(jnp.bfloat16)
    sin = jnp.sin(ang).astype(jnp.bfloat16)

    out = _fused_attn_allreduce(
        xb, _deint(Wq, Hl, scale=SCALE), _deint(Wk, Hl),
        Wv.astype(jnp.bfloat16), Wo.astype(jnp.bfloat16),
        cos, sin, B, Hl)
    return out.astype(jnp.float32).reshape(B, Sq, D)
